# baseline (speedup 1.0000x reference)
import jax
import jax.numpy as jnp
from jax.experimental import pallas as pl
from jax.experimental.pallas import tpu as pltpu

NQ = 6
K = 1024
D = 256
BATCH = 8
T = 256
NTOK = BATCH * T
CHUNK = 512
NCHUNK = NTOK // CHUNK
DN = (((1,), (1,)), ((), ()))
DNG = (((1,), (0,)), ((), ()))


def _vq(x, C, Chi, Cmid, Clo):
    cn = jnp.sum(C * C, axis=1)[None, :]
    qs, idxs = [], []
    counts = jnp.zeros((1, K), jnp.float32)
    commit = jnp.float32(0.0)
    for c in range(NCHUNK):
        xc = jax.lax.slice(x, (c * CHUNK, 0), ((c + 1) * CHUNK, D))
        S2 = jax.lax.dot_general((xc + xc).astype(jnp.bfloat16), Chi, DN,
                                 preferred_element_type=jnp.float32)
        xn = jnp.sum(xc * xc, axis=1, keepdims=True)
        dist = xn - S2 + cn
        m = jnp.min(dist, axis=1, keepdims=True)
        lanes = jax.lax.broadcasted_iota(jnp.int32, (CHUNK, K), 1)
        idxc = jnp.min(jnp.where(dist <= m, lanes, K), axis=1)
        ohb = (lanes == idxc[:, None]).astype(jnp.bfloat16)
        xd = ((jax.lax.dot_general(ohb, Chi, DNG, preferred_element_type=jnp.float32)
               + jax.lax.dot_general(ohb, Cmid, DNG, preferred_element_type=jnp.float32))
              + jax.lax.dot_general(ohb, Clo, DNG, preferred_element_type=jnp.float32))
        qs.append(xc + (xd - xc))
        idxs.append(idxc)
        commit = commit + jnp.sum(m)
        counts = counts + jnp.sum(ohb, axis=0, keepdims=True,
                                  dtype=jnp.float32)
    q = jnp.concatenate(qs, axis=0)
    idx = jnp.concatenate(idxs, axis=0)
    prob = counts / NTOK
    perp = jnp.exp(-jnp.sum(prob * jnp.log(prob + 1e-7)))
    commit_loss = commit / (NTOK * D)
    return q, idx, commit_loss, perp


def _transform(qtok, W, tb):
    ys = []
    Wbf = W.astype(jnp.bfloat16)
    for bi in range(BATCH):
        Xb = jax.lax.slice(qtok, (bi * T, 0), ((bi + 1) * T, D))
        yb = jax.lax.dot_general(Wbf, Xb.astype(jnp.bfloat16), DN,
                                 preferred_element_type=jnp.float32)
        ys.append(yb + tb)
    return jnp.concatenate(ys, axis=0)


def _conv(y, r, cw_ref, cb):
    u = jnp.concatenate([y, r], axis=1).astype(jnp.bfloat16)
    p0 = jax.lax.dot_general(u, cw_ref[0].astype(jnp.bfloat16), DN,
                             preferred_element_type=jnp.float32)
    p1 = jax.lax.dot_general(u, cw_ref[1].astype(jnp.bfloat16), DN,
                             preferred_element_type=jnp.float32)
    p2 = jax.lax.dot_general(u, cw_ref[2].astype(jnp.bfloat16), DN,
                             preferred_element_type=jnp.float32)
    rows = jax.lax.broadcasted_iota(jnp.int32, (NTOK, 1), 0)
    zrow = jnp.zeros((1, D), jnp.float32)
    p0s = jnp.concatenate([zrow, jax.lax.slice(p0, (0, 0), (NTOK - 1, D))], axis=0)
    p0s = jnp.where(rows % T == 0, 0.0, p0s)
    p2s = jnp.concatenate([jax.lax.slice(p2, (1, 0), (NTOK, D)), zrow], axis=0)
    p2s = jnp.where(rows % T == T - 1, 0.0, p2s)
    return (p0s + p1) + p2s + cb


def _body(xb_ref, xh_ref, xf_ref, cbb_ref, cbh_ref, cbf_ref,
          hib_ref, midb_ref, lob_ref, hih_ref, midh_ref, loh_ref,
          hif_ref, midf_ref, lof_ref,
          W_ref, tb_ref, cw_ref, cbias_ref,
          qob_ref, qoh_ref, qof_ref, idx_ref, loss_ref, perp_ref,
          rb_ref, rh_ref, rf_ref):
    qi = pl.program_id(0)

    @pl.when(qi == 0)
    def _init():
        rb_ref[...] = xb_ref[...]
        rh_ref[...] = xh_ref[...]
        rf_ref[...] = xf_ref[...]
        qob_ref[...] = jnp.zeros_like(qob_ref)
        qoh_ref[...] = jnp.zeros_like(qoh_ref)
        qof_ref[...] = jnp.zeros_like(qof_ref)
        loss_ref[...] = jnp.zeros_like(loss_ref)
        perp_ref[...] = jnp.zeros_like(perp_ref)

    W = W_ref[...]
    tb = tb_ref[...]
    cbias = cbias_ref[...]

    xb = rb_ref[...]
    qb, ib, lb, pb = _vq(xb, cbb_ref[0], hib_ref[0], midb_ref[0], lob_ref[0])
    rb_ref[...] = xb - qb
    qob_ref[...] = qob_ref[...] + qb
    idx_ref[0, 0, :] = ib

    hh = _conv(_transform(qb, W, tb), rh_ref[...], cw_ref, cbias)
    qh, ih, lh, ph = _vq(hh, cbh_ref[0], hih_ref[0], midh_ref[0], loh_ref[0])
    rh_ref[...] = rh_ref[...] - qh
    qoh_ref[...] = qoh_ref[...] + qh
    idx_ref[0, 1, :] = ih

    hf = _conv(_transform(qh, W, tb), rf_ref[...], cw_ref, cbias)
    qf, iff, lf, pf = _vq(hf, cbf_ref[0], hif_ref[0], midf_ref[0], lof_ref[0])
    rf_ref[...] = rf_ref[...] - qf
    qof_ref[...] = qof_ref[...] + qf
    idx_ref[0, 2, :] = iff

    loss_ref[...] = loss_ref[...] + (lb + lh + lf)
    perp_ref[...] = perp_ref[...] + (pb + ph + pf)

    @pl.when(qi == NQ - 1)
    def _fin():
        loss_ref[...] = loss_ref[...] / 6.0
        perp_ref[...] = perp_ref[...] / 6.0


def kernel(x_body, x_hands, x_face, codebooks_body, codebooks_hands,
           codebooks_face, trans_W, trans_b, conv_w, conv_b):
    xb = jnp.transpose(x_body, (0, 2, 1)).reshape(NTOK, D)
    xh = jnp.transpose(x_hands, (0, 2, 1)).reshape(NTOK, D)
    xf = jnp.transpose(x_face, (0, 2, 1)).reshape(NTOK, D)
    cw = jnp.transpose(conv_w, (2, 0, 1))
    tb = trans_b.reshape(D, 1)
    cb = conv_b.reshape(1, D)

    def split3(C):
        hi = C.astype(jnp.bfloat16)
        r1 = C - hi.astype(jnp.float32)
        mid = r1.astype(jnp.bfloat16)
        lo = (r1 - mid.astype(jnp.float32)).astype(jnp.bfloat16)
        return hi, mid, lo

    hib, midb, lob = split3(codebooks_body)
    hih, midh, loh = split3(codebooks_hands)
    hif, midf, lof = split3(codebooks_face)

    full2 = lambda s: pl.BlockSpec(s, lambda q: (0, 0))
    full3 = lambda s: pl.BlockSpec(s, lambda q: (0, 0, 0))
    cbspec = pl.BlockSpec((1, K, D), lambda q: (q, 0, 0))

    qo_b, qo_h, qo_f, idx, loss_o, perp_o = pl.pallas_call(
        _body,
        grid=(NQ,),
        in_specs=[full2((NTOK, D)), full2((NTOK, D)), full2((NTOK, D)),
                  cbspec, cbspec, cbspec,
                  cbspec, cbspec, cbspec, cbspec, cbspec, cbspec,
                  cbspec, cbspec, cbspec,
                  full2((D, D)), full2((D, 1)), full3((3, D, 2 * D)),
                  full2((1, D))],
        out_specs=[full2((NTOK, D)), full2((NTOK, D)), full2((NTOK, D)),
                   pl.BlockSpec((1, 3, NTOK), lambda q: (q, 0, 0)),
                   full2((8, 128)), full2((8, 128))],
        out_shape=[jax.ShapeDtypeStruct((NTOK, D), jnp.float32),
                   jax.ShapeDtypeStruct((NTOK, D), jnp.float32),
                   jax.ShapeDtypeStruct((NTOK, D), jnp.float32),
                   jax.ShapeDtypeStruct((NQ, 3, NTOK), jnp.int32),
                   jax.ShapeDtypeStruct((8, 128), jnp.float32),
                   jax.ShapeDtypeStruct((8, 128), jnp.float32)],
        scratch_shapes=[pltpu.VMEM((NTOK, D), jnp.float32)] * 3,
        compiler_params=pltpu.CompilerParams(
            dimension_semantics=("arbitrary",)),
    )(xb, xh, xf, codebooks_body, codebooks_hands, codebooks_face,
      hib, midb, lob, hih, midh, loh, hif, midf, lof,
      trans_W, tb, cw, cb)

    tomajor = lambda q: jnp.transpose(q.reshape(BATCH, T, D), (0, 2, 1))
    quantized_out = jnp.concatenate(
        [tomajor(qo_b), tomajor(qo_h), tomajor(qo_f)], axis=1)
    all_indices = (idx.reshape(NQ, 3, BATCH, T)
                   .transpose(2, 1, 3, 0).reshape(BATCH, 3 * T, NQ))
    return quantized_out, all_indices, loss_o[0, 0], perp_o[0, 0]


# speedup vs baseline: 1.0886x; 1.0886x over previous
import jax
import jax.numpy as jnp
from jax.experimental import pallas as pl
from jax.experimental.pallas import tpu as pltpu

NQ = 6
K = 1024
D = 256
BATCH = 8
T = 256
NTOK = BATCH * T
CHUNK = 512
NCHUNK = NTOK // CHUNK
DN = (((1,), (1,)), ((), ()))
DNG = (((1,), (0,)), ((), ()))


def _vq(x, C):
    cn = jnp.sum(C * C, axis=1)[None, :]
    Chi = C.astype(jnp.bfloat16)
    r1 = C - Chi.astype(jnp.float32)
    Cmid = r1.astype(jnp.bfloat16)
    Clo = (r1 - Cmid.astype(jnp.float32)).astype(jnp.bfloat16)
    qs, idxs = [], []
    counts = jnp.zeros((1, K), jnp.float32)
    commit = jnp.float32(0.0)
    for c in range(NCHUNK):
        xc = jax.lax.slice(x, (c * CHUNK, 0), ((c + 1) * CHUNK, D))
        S2 = jax.lax.dot_general((xc + xc).astype(jnp.bfloat16), Chi, DN,
                                 preferred_element_type=jnp.float32)
        xn = jnp.sum(xc * xc, axis=1, keepdims=True)
        dist = xn - S2 + cn
        m = jnp.min(dist, axis=1, keepdims=True)
        lanes = jax.lax.broadcasted_iota(jnp.int32, (CHUNK, K), 1)
        idxc = jnp.min(jnp.where(dist <= m, lanes, K), axis=1)
        ohb = (lanes == idxc[:, None]).astype(jnp.bfloat16)
        xd = ((jax.lax.dot_general(ohb, Chi, DNG, preferred_element_type=jnp.float32)
               + jax.lax.dot_general(ohb, Cmid, DNG, preferred_element_type=jnp.float32))
              + jax.lax.dot_general(ohb, Clo, DNG, preferred_element_type=jnp.float32))
        qs.append(xc + (xd - xc))
        idxs.append(idxc)
        commit = commit + jnp.sum(m)
        counts = counts + jnp.sum(ohb, axis=0, keepdims=True,
                                  dtype=jnp.float32)
    q = jnp.concatenate(qs, axis=0)
    idx = jnp.concatenate(idxs, axis=0)
    prob = counts / NTOK
    perp = jnp.exp(-jnp.sum(prob * jnp.log(prob + 1e-7)))
    commit_loss = commit / (NTOK * D)
    return q, idx, commit_loss, perp


def _transform(qtok, W, tb):
    ys = []
    Wbf = W.astype(jnp.bfloat16)
    for bi in range(BATCH):
        Xb = jax.lax.slice(qtok, (bi * T, 0), ((bi + 1) * T, D))
        yb = jax.lax.dot_general(Wbf, Xb.astype(jnp.bfloat16), DN,
                                 preferred_element_type=jnp.float32)
        ys.append(yb + tb)
    return jnp.concatenate(ys, axis=0)


def _conv(y, r, cw_ref, cb):
    u = jnp.concatenate([y, r], axis=1).astype(jnp.bfloat16)
    p0 = jax.lax.dot_general(u, cw_ref[0].astype(jnp.bfloat16), DN,
                             preferred_element_type=jnp.float32)
    p1 = jax.lax.dot_general(u, cw_ref[1].astype(jnp.bfloat16), DN,
                             preferred_element_type=jnp.float32)
    p2 = jax.lax.dot_general(u, cw_ref[2].astype(jnp.bfloat16), DN,
                             preferred_element_type=jnp.float32)
    rows = jax.lax.broadcasted_iota(jnp.int32, (NTOK, 1), 0)
    zrow = jnp.zeros((1, D), jnp.float32)
    p0s = jnp.concatenate([zrow, jax.lax.slice(p0, (0, 0), (NTOK - 1, D))], axis=0)
    p0s = jnp.where(rows % T == 0, 0.0, p0s)
    p2s = jnp.concatenate([jax.lax.slice(p2, (1, 0), (NTOK, D)), zrow], axis=0)
    p2s = jnp.where(rows % T == T - 1, 0.0, p2s)
    return (p0s + p1) + p2s + cb


def _body(xb_ref, xh_ref, xf_ref, cbb_ref, cbh_ref, cbf_ref,
          W_ref, tb_ref, cw_ref, cbias_ref,
          qob_ref, qoh_ref, qof_ref, idx_ref, loss_ref, perp_ref,
          rb_ref, rh_ref, rf_ref):
    qi = pl.program_id(0)

    @pl.when(qi == 0)
    def _init():
        rb_ref[...] = xb_ref[...]
        rh_ref[...] = xh_ref[...]
        rf_ref[...] = xf_ref[...]
        qob_ref[...] = jnp.zeros_like(qob_ref)
        qoh_ref[...] = jnp.zeros_like(qoh_ref)
        qof_ref[...] = jnp.zeros_like(qof_ref)
        loss_ref[...] = jnp.zeros_like(loss_ref)
        perp_ref[...] = jnp.zeros_like(perp_ref)

    W = W_ref[...]
    tb = tb_ref[...]
    cbias = cbias_ref[...]

    xb = rb_ref[...]
    qb, ib, lb, pb = _vq(xb, cbb_ref[0])
    rb_ref[...] = xb - qb
    qob_ref[...] = qob_ref[...] + qb
    idx_ref[0, 0, :] = ib

    hh = _conv(_transform(qb, W, tb), rh_ref[...], cw_ref, cbias)
    qh, ih, lh, ph = _vq(hh, cbh_ref[0])
    rh_ref[...] = rh_ref[...] - qh
    qoh_ref[...] = qoh_ref[...] + qh
    idx_ref[0, 1, :] = ih

    hf = _conv(_transform(qh, W, tb), rf_ref[...], cw_ref, cbias)
    qf, iff, lf, pf = _vq(hf, cbf_ref[0])
    rf_ref[...] = rf_ref[...] - qf
    qof_ref[...] = qof_ref[...] + qf
    idx_ref[0, 2, :] = iff

    loss_ref[...] = loss_ref[...] + (lb + lh + lf)
    perp_ref[...] = perp_ref[...] + (pb + ph + pf)

    @pl.when(qi == NQ - 1)
    def _fin():
        loss_ref[...] = loss_ref[...] / 6.0
        perp_ref[...] = perp_ref[...] / 6.0


def kernel(x_body, x_hands, x_face, codebooks_body, codebooks_hands,
           codebooks_face, trans_W, trans_b, conv_w, conv_b):
    xb = jnp.transpose(x_body, (0, 2, 1)).reshape(NTOK, D)
    xh = jnp.transpose(x_hands, (0, 2, 1)).reshape(NTOK, D)
    xf = jnp.transpose(x_face, (0, 2, 1)).reshape(NTOK, D)
    cw = jnp.transpose(conv_w, (2, 0, 1))
    tb = trans_b.reshape(D, 1)
    cb = conv_b.reshape(1, D)

    full2 = lambda s: pl.BlockSpec(s, lambda q: (0, 0))
    full3 = lambda s: pl.BlockSpec(s, lambda q: (0, 0, 0))
    cbspec = pl.BlockSpec((1, K, D), lambda q: (q, 0, 0))

    qo_b, qo_h, qo_f, idx, loss_o, perp_o = pl.pallas_call(
        _body,
        grid=(NQ,),
        in_specs=[full2((NTOK, D)), full2((NTOK, D)), full2((NTOK, D)),
                  cbspec, cbspec, cbspec,
                  full2((D, D)), full2((D, 1)), full3((3, D, 2 * D)),
                  full2((1, D))],
        out_specs=[full2((NTOK, D)), full2((NTOK, D)), full2((NTOK, D)),
                   pl.BlockSpec((1, 3, NTOK), lambda q: (q, 0, 0)),
                   full2((8, 128)), full2((8, 128))],
        out_shape=[jax.ShapeDtypeStruct((NTOK, D), jnp.float32),
                   jax.ShapeDtypeStruct((NTOK, D), jnp.float32),
                   jax.ShapeDtypeStruct((NTOK, D), jnp.float32),
                   jax.ShapeDtypeStruct((NQ, 3, NTOK), jnp.int32),
                   jax.ShapeDtypeStruct((8, 128), jnp.float32),
                   jax.ShapeDtypeStruct((8, 128), jnp.float32)],
        scratch_shapes=[pltpu.VMEM((NTOK, D), jnp.float32)] * 3,
        compiler_params=pltpu.CompilerParams(
            dimension_semantics=("arbitrary",)),
    )(xb, xh, xf, codebooks_body, codebooks_hands, codebooks_face,
      trans_W, tb, cw, cb)

    tomajor = lambda q: jnp.transpose(q.reshape(BATCH, T, D), (0, 2, 1))
    quantized_out = jnp.concatenate(
        [tomajor(qo_b), tomajor(qo_h), tomajor(qo_f)], axis=1)
    all_indices = (idx.reshape(NQ, 3, BATCH, T)
                   .transpose(2, 1, 3, 0).reshape(BATCH, 3 * T, NQ))
    return quantized_out, all_indices, loss_o[0, 0], perp_o[0, 0]


# speedup vs baseline: 1.0904x; 1.0017x over previous
import jax
import jax.numpy as jnp
from jax.experimental import pallas as pl
from jax.experimental.pallas import tpu as pltpu

NQ = 6
K = 1024
D = 256
BATCH = 8
T = 256
NTOK = BATCH * T
CHUNK = 512
NCHUNK = NTOK // CHUNK
DN = (((1,), (1,)), ((), ()))
DNG = (((1,), (0,)), ((), ()))


def _vq(x, C):
    cn = jnp.sum(C * C, axis=1)[None, :]
    Chi = C.astype(jnp.bfloat16)
    r1 = C - Chi.astype(jnp.float32)
    Cmid = r1.astype(jnp.bfloat16)
    Clo = (r1 - Cmid.astype(jnp.float32)).astype(jnp.bfloat16)
    qs, idxs = [], []
    counts = jnp.zeros((1, K), jnp.float32)
    commit = jnp.float32(0.0)
    for c in range(NCHUNK):
        xc = jax.lax.slice(x, (c * CHUNK, 0), ((c + 1) * CHUNK, D))
        S2 = jax.lax.dot_general((xc + xc).astype(jnp.bfloat16), Chi, DN,
                                 preferred_element_type=jnp.float32)
        xn = jnp.sum(xc * xc, axis=1, keepdims=True)
        dist = xn - S2 + cn
        m = jnp.min(dist, axis=1, keepdims=True)
        lanes = jax.lax.broadcasted_iota(jnp.int32, (CHUNK, K), 1)
        idxc = jnp.min(jnp.where(dist <= m, lanes, K), axis=1)
        ohb = (lanes == idxc[:, None]).astype(jnp.bfloat16)
        xd = ((jax.lax.dot_general(ohb, Chi, DNG, preferred_element_type=jnp.float32)
               + jax.lax.dot_general(ohb, Cmid, DNG, preferred_element_type=jnp.float32))
              + jax.lax.dot_general(ohb, Clo, DNG, preferred_element_type=jnp.float32))
        qs.append(xc + (xd - xc))
        idxs.append(idxc)
        commit = commit + jnp.sum(m)
        counts = counts + jnp.sum(ohb, axis=0, keepdims=True,
                                  dtype=jnp.float32)
    q = jnp.concatenate(qs, axis=0)
    idx = jnp.concatenate(idxs, axis=0)
    prob = counts / NTOK
    perp = jnp.exp(-jnp.sum(prob * jnp.log(prob + 1e-7)))
    commit_loss = commit / (NTOK * D)
    return q, idx, commit_loss, perp


def _transform(qtok, W, tb):
    Yall = jax.lax.dot_general(W.astype(jnp.bfloat16),
                               qtok.astype(jnp.bfloat16), DN,
                               preferred_element_type=jnp.float32)
    Yall = Yall + tb
    ys = []
    for bi in range(BATCH):
        ys.append(jax.lax.slice(Yall, (0, bi * T), (D, (bi + 1) * T)))
    return jnp.concatenate(ys, axis=0)


def _conv(y, r, cw_ref, cb):
    u = jnp.concatenate([y, r], axis=1).astype(jnp.bfloat16)
    p0 = jax.lax.dot_general(u, cw_ref[0].astype(jnp.bfloat16), DN,
                             preferred_element_type=jnp.float32)
    p1 = jax.lax.dot_general(u, cw_ref[1].astype(jnp.bfloat16), DN,
                             preferred_element_type=jnp.float32)
    p2 = jax.lax.dot_general(u, cw_ref[2].astype(jnp.bfloat16), DN,
                             preferred_element_type=jnp.float32)
    rows = jax.lax.broadcasted_iota(jnp.int32, (NTOK, 1), 0)
    zrow = jnp.zeros((1, D), jnp.float32)
    p0s = jnp.concatenate([zrow, jax.lax.slice(p0, (0, 0), (NTOK - 1, D))], axis=0)
    p0s = jnp.where(rows % T == 0, 0.0, p0s)
    p2s = jnp.concatenate([jax.lax.slice(p2, (1, 0), (NTOK, D)), zrow], axis=0)
    p2s = jnp.where(rows % T == T - 1, 0.0, p2s)
    return (p0s + p1) + p2s + cb


def _body(xb_ref, xh_ref, xf_ref, cbb_ref, cbh_ref, cbf_ref,
          W_ref, tb_ref, cw_ref, cbias_ref,
          qob_ref, qoh_ref, qof_ref, idx_ref, loss_ref, perp_ref,
          rb_ref, rh_ref, rf_ref):
    qi = pl.program_id(0)

    @pl.when(qi == 0)
    def _init():
        rb_ref[...] = xb_ref[...]
        rh_ref[...] = xh_ref[...]
        rf_ref[...] = xf_ref[...]
        qob_ref[...] = jnp.zeros_like(qob_ref)
        qoh_ref[...] = jnp.zeros_like(qoh_ref)
        qof_ref[...] = jnp.zeros_like(qof_ref)
        loss_ref[...] = jnp.zeros_like(loss_ref)
        perp_ref[...] = jnp.zeros_like(perp_ref)

    W = W_ref[...]
    tb = tb_ref[...]
    cbias = cbias_ref[...]

    xb = rb_ref[...]
    qb, ib, lb, pb = _vq(xb, cbb_ref[0])
    rb_ref[...] = xb - qb
    qob_ref[...] = qob_ref[...] + qb
    idx_ref[0, 0, :] = ib

    hh = _conv(_transform(qb, W, tb), rh_ref[...], cw_ref, cbias)
    qh, ih, lh, ph = _vq(hh, cbh_ref[0])
    rh_ref[...] = rh_ref[...] - qh
    qoh_ref[...] = qoh_ref[...] + qh
    idx_ref[0, 1, :] = ih

    hf = _conv(_transform(qh, W, tb), rf_ref[...], cw_ref, cbias)
    qf, iff, lf, pf = _vq(hf, cbf_ref[0])
    rf_ref[...] = rf_ref[...] - qf
    qof_ref[...] = qof_ref[...] + qf
    idx_ref[0, 2, :] = iff

    loss_ref[...] = loss_ref[...] + (lb + lh + lf)
    perp_ref[...] = perp_ref[...] + (pb + ph + pf)

    @pl.when(qi == NQ - 1)
    def _fin():
        loss_ref[...] = loss_ref[...] / 6.0
        perp_ref[...] = perp_ref[...] / 6.0


def kernel(x_body, x_hands, x_face, codebooks_body, codebooks_hands,
           codebooks_face, trans_W, trans_b, conv_w, conv_b):
    xb = jnp.transpose(x_body, (0, 2, 1)).reshape(NTOK, D)
    xh = jnp.transpose(x_hands, (0, 2, 1)).reshape(NTOK, D)
    xf = jnp.transpose(x_face, (0, 2, 1)).reshape(NTOK, D)
    cw = jnp.transpose(conv_w, (2, 0, 1))
    tb = trans_b.reshape(D, 1)
    cb = conv_b.reshape(1, D)

    full2 = lambda s: pl.BlockSpec(s, lambda q: (0, 0))
    full3 = lambda s: pl.BlockSpec(s, lambda q: (0, 0, 0))
    cbspec = pl.BlockSpec((1, K, D), lambda q: (q, 0, 0))

    qo_b, qo_h, qo_f, idx, loss_o, perp_o = pl.pallas_call(
        _body,
        grid=(NQ,),
        in_specs=[full2((NTOK, D)), full2((NTOK, D)), full2((NTOK, D)),
                  cbspec, cbspec, cbspec,
                  full2((D, D)), full2((D, 1)), full3((3, D, 2 * D)),
                  full2((1, D))],
        out_specs=[full2((NTOK, D)), full2((NTOK, D)), full2((NTOK, D)),
                   pl.BlockSpec((1, 3, NTOK), lambda q: (q, 0, 0)),
                   full2((8, 128)), full2((8, 128))],
        out_shape=[jax.ShapeDtypeStruct((NTOK, D), jnp.float32),
                   jax.ShapeDtypeStruct((NTOK, D), jnp.float32),
                   jax.ShapeDtypeStruct((NTOK, D), jnp.float32),
                   jax.ShapeDtypeStruct((NQ, 3, NTOK), jnp.int32),
                   jax.ShapeDtypeStruct((8, 128), jnp.float32),
                   jax.ShapeDtypeStruct((8, 128), jnp.float32)],
        scratch_shapes=[pltpu.VMEM((NTOK, D), jnp.float32)] * 3,
        compiler_params=pltpu.CompilerParams(
            dimension_semantics=("arbitrary",)),
    )(xb, xh, xf, codebooks_body, codebooks_hands, codebooks_face,
      trans_W, tb, cw, cb)

    tomajor = lambda q: jnp.transpose(q.reshape(BATCH, T, D), (0, 2, 1))
    quantized_out = jnp.concatenate(
        [tomajor(qo_b), tomajor(qo_h), tomajor(qo_f)], axis=1)
    all_indices = (idx.reshape(NQ, 3, BATCH, T)
                   .transpose(2, 1, 3, 0).reshape(BATCH, 3 * T, NQ))
    return quantized_out, all_indices, loss_o[0, 0], perp_o[0, 0]
